# baseline (speedup 1.0000x reference)
import functools

import jax
import jax.numpy as jnp
from jax import lax
from jax.experimental import pallas as pl
from jax.experimental.pallas import tpu as pltpu
from jax.experimental.pallas import tpu_sc as plsc

B, S = 4, 8192
V, D = 256, 1024
N = B * S

NC, NS = 2, 16
NW = NC * NS
NQ = 4
DQ = D // NQ
NG = NW // NQ
ROWS_PER_G = N // NG
C = 64
NCHUNK = ROWS_PER_G // C

_mesh = plsc.VectorSubcoreMesh(core_axis_name="c", subcore_axis_name="s")


@functools.partial(
    pl.kernel,
    mesh=_mesh,
    compiler_params=pltpu.CompilerParams(use_tc_tiling_on_sc=False,
                                         needs_layout_passes=False),
    out_type=jax.ShapeDtypeStruct((N, D), jnp.float32),
    scratch_types=[
        pltpu.VMEM((NCHUNK, C), jnp.int32),
        pltpu.VMEM((V * DQ,), jnp.float32),
        pltpu.VMEM((C, DQ), jnp.float32),
        pltpu.VMEM((C, DQ), jnp.float32),
        pltpu.SemaphoreType.DMA,
        pltpu.SemaphoreType.DMA,
    ],
)
def _sc_gather(idx_hbm, shards_hbm, out_hbm, idx_v, table_v, buf0, buf1,
               wsem0, wsem1):
    wid = lax.axis_index("s") * NC + lax.axis_index("c")
    q = wid % NQ
    g = wid // NQ

    pltpu.sync_copy(shards_hbm.at[q], table_v)
    pltpu.sync_copy(idx_hbm.at[g], idx_v)

    bufs = (buf0, buf1)
    wsems = (wsem0, wsem1)
    col0 = q * DQ

    def copy_chunk(c, buf):
        for gi in range(C // 16):
            src0 = idx_v[c, pl.ds(gi * 16, 16)] * DQ
            pos_vec = gi * 16 + lax.broadcasted_iota(jnp.int32, (16,), 0)

            def _body(j, col, src0=src0, pos_vec=pos_vec):
                vals = plsc.load_gather(table_v, [src0 + col])
                plsc.store_scatter(buf, [pos_vec, col], vals)
                return col + 1
            lax.fori_loop(0, DQ, _body,
                          jnp.zeros((16,), jnp.int32))

    def wait_write(buf, wsem):
        pltpu.make_async_copy(
            buf, out_hbm.at[pl.ds(0, C), pl.ds(col0, DQ)], wsem).wait()

    def outer(i2, carry):
        for b in range(2):
            c = i2 * 2 + b

            @pl.when(c >= 2)
            def _():
                wait_write(bufs[b], wsems[b])

            copy_chunk(c, bufs[b])
            row0 = g * ROWS_PER_G + c * C
            pltpu.async_copy(
                bufs[b], out_hbm.at[pl.ds(row0, C), pl.ds(col0, DQ)],
                wsems[b])
        return carry

    lax.fori_loop(0, NCHUNK // 2, outer, 0)
    wait_write(buf0, wsem0)
    wait_write(buf1, wsem1)


def kernel(input_ids, attention_mask, embed):
    idx = input_ids.reshape(NG, NCHUNK, C).astype(jnp.int32)
    shards = embed.reshape(V, NQ, DQ).transpose(1, 0, 2).reshape(NQ, V * DQ)
    out = _sc_gather(idx, shards)
    return out.reshape(B, S, D)


# speedup vs baseline: 5.6678x; 5.6678x over previous
import functools

import jax
import jax.numpy as jnp
from jax import lax
from jax.experimental import pallas as pl
from jax.experimental.pallas import tpu as pltpu
from jax.experimental.pallas import tpu_sc as plsc

B, S = 4, 8192
V, D = 256, 1024
N = B * S

NC, NS = 2, 16
NW = NC * NS
NQ = 4
DQ = D // NQ
NG = NW // NQ
ROWS_PER_G = N // NG
C = 64
NCHUNK = ROWS_PER_G // C

_mesh = plsc.VectorSubcoreMesh(core_axis_name="c", subcore_axis_name="s")


@functools.partial(
    pl.kernel,
    mesh=_mesh,
    compiler_params=pltpu.CompilerParams(use_tc_tiling_on_sc=False,
                                         needs_layout_passes=False),
    out_type=jax.ShapeDtypeStruct((N, D), jnp.float32),
    scratch_types=[
        pltpu.VMEM((NCHUNK, C), jnp.int32),
        pltpu.VMEM((V, DQ), jnp.float32),
        pltpu.VMEM((C, DQ), jnp.float32),
        pltpu.VMEM((C, DQ), jnp.float32),
        pltpu.SemaphoreType.DMA,
        pltpu.SemaphoreType.DMA,
    ],
)
def _sc_gather(idx_hbm, shards_hbm, out_hbm, idx_v, table_v, buf0, buf1,
               wsem0, wsem1):
    wid = lax.axis_index("s") * NC + lax.axis_index("c")
    q = wid % NQ
    g = wid // NQ

    pltpu.sync_copy(shards_hbm.at[q], table_v)
    pltpu.sync_copy(idx_hbm.at[g], idx_v)

    bufs = (buf0, buf1)
    wsems = (wsem0, wsem1)
    col0 = q * DQ

    def copy_chunk(c, buf):
        def one_group(gi, carry):
            vec = idx_v[c, pl.ds(gi * 16, 16)]
            for l in range(16):
                r = vec[l]
                p = gi * 16 + l
                vals = [table_v[r, pl.ds(j * 16, 16)]
                        for j in range(DQ // 16)]
                for j in range(DQ // 16):
                    buf[p, pl.ds(j * 16, 16)] = vals[j]
            return carry
        lax.fori_loop(0, C // 16, one_group, 0)

    def wait_write(buf, wsem):
        pltpu.make_async_copy(
            buf, out_hbm.at[pl.ds(0, C), pl.ds(col0, DQ)], wsem).wait()

    def outer(i2, carry):
        for b in range(2):
            c = i2 * 2 + b

            @pl.when(c >= 2)
            def _():
                wait_write(bufs[b], wsems[b])

            copy_chunk(c, bufs[b])
            row0 = g * ROWS_PER_G + c * C
            pltpu.async_copy(
                bufs[b], out_hbm.at[pl.ds(row0, C), pl.ds(col0, DQ)],
                wsems[b])
        return carry

    lax.fori_loop(0, NCHUNK // 2, outer, 0)
    wait_write(buf0, wsem0)
    wait_write(buf1, wsem1)


def kernel(input_ids, attention_mask, embed):
    idx = input_ids.reshape(NG, NCHUNK, C).astype(jnp.int32)
    shards = embed.reshape(V, NQ, DQ).transpose(1, 0, 2)
    out = _sc_gather(idx, shards)
    return out.reshape(B, S, D)


# speedup vs baseline: 6.4158x; 1.1320x over previous
import functools

import jax
import jax.numpy as jnp
from jax import lax
from jax.experimental import pallas as pl
from jax.experimental.pallas import tpu as pltpu
from jax.experimental.pallas import tpu_sc as plsc

B, S = 4, 8192
V, D = 256, 1024
N = B * S

NC, NS = 2, 16
NW = NC * NS
ROWS_PER_W = N // NW
C = 16
NCHUNK = ROWS_PER_W // C
CSPLIT = 704
CCOMP = D - CSPLIT

_mesh = plsc.VectorSubcoreMesh(core_axis_name="c", subcore_axis_name="s")


@functools.partial(
    pl.kernel,
    mesh=_mesh,
    compiler_params=pltpu.CompilerParams(use_tc_tiling_on_sc=False,
                                         needs_layout_passes=False),
    out_type=jax.ShapeDtypeStruct((N, D), jnp.float32),
    scratch_types=[
        pltpu.VMEM((NCHUNK, C), jnp.int32),
        pltpu.VMEM((V, CCOMP), jnp.float32),
        pltpu.VMEM((C, D), jnp.float32),
        pltpu.VMEM((C, D), jnp.float32),
        pltpu.VMEM_SHARED((V, CSPLIT), jnp.float32),
        pltpu.SemaphoreType.DMA,
        pltpu.SemaphoreType.DMA,
        pltpu.SemaphoreType.DMA,
        pltpu.SemaphoreType.DMA,
    ],
)
def _sc_gather(idx_hbm, front_hbm, shard_hbm, out_hbm, idx_v, shard_v,
               buf0, buf1, table_sh, sem0, sem1, wsem0, wsem1):
    sid = lax.axis_index("s")
    wid = sid * NC + lax.axis_index("c")

    @pl.when(sid == 0)
    def _():
        pltpu.sync_copy(front_hbm, table_sh)

    pltpu.sync_copy(shard_hbm, shard_v)
    pltpu.sync_copy(idx_hbm.at[wid], idx_v)
    plsc.subcore_barrier()

    base = wid * ROWS_PER_W
    bufs = (buf0, buf1)
    sems = (sem0, sem1)
    wsems = (wsem0, wsem1)

    def fill_chunk(c, buf, sem):
        vec = idx_v[c, pl.ds(0, C)]
        for l in range(C):
            r = vec[l]
            pltpu.async_copy(table_sh.at[r],
                             buf.at[l, pl.ds(0, CSPLIT)], sem)
        for l in range(C):
            r = vec[l]
            vals = [shard_v[r, pl.ds(j * 16, 16)]
                    for j in range(CCOMP // 16)]
            for j in range(CCOMP // 16):
                buf[l, pl.ds(CSPLIT + j * 16, 16)] = vals[j]

    def wait_streams(buf, sem):
        pltpu.make_async_copy(table_sh.at[pl.ds(0, C)],
                              buf.at[pl.ds(0, C), pl.ds(0, CSPLIT)],
                              sem).wait()

    def wait_write(buf, wsem):
        pltpu.make_async_copy(buf, out_hbm.at[pl.ds(0, C)], wsem).wait()

    def outer(i2, carry):
        for b in range(2):
            c = i2 * 2 + b

            @pl.when(c >= 2)
            def _():
                wait_write(bufs[b], wsems[b])

            fill_chunk(c, bufs[b], sems[b])
            wait_streams(bufs[b], sems[b])
            pltpu.async_copy(bufs[b], out_hbm.at[pl.ds(base + c * C, C)],
                             wsems[b])
        return carry

    lax.fori_loop(0, NCHUNK // 2, outer, 0)
    wait_write(buf0, wsem0)
    wait_write(buf1, wsem1)


def kernel(input_ids, attention_mask, embed):
    idx = input_ids.reshape(NW, NCHUNK, C).astype(jnp.int32)
    front = embed[:, :CSPLIT]
    shard = embed[:, CSPLIT:]
    out = _sc_gather(idx, front, shard)
    return out.reshape(B, S, D)


# speedup vs baseline: 13.4407x; 2.0949x over previous
import functools

import jax
import jax.numpy as jnp
from jax import lax
from jax.experimental import pallas as pl
from jax.experimental.pallas import tpu as pltpu
from jax.experimental.pallas import tpu_sc as plsc

B, S = 4, 8192
V, D = 256, 1024
N = B * S

NC, NS = 2, 16
NW = NC * NS
ROWS_PER_W = N // NW
C = 32
NCHUNK = ROWS_PER_W // C

_mesh = plsc.VectorSubcoreMesh(core_axis_name="c", subcore_axis_name="s")


@functools.partial(
    pl.kernel,
    mesh=_mesh,
    out_type=jax.ShapeDtypeStruct((N, D), jnp.float32),
    scratch_types=[
        pltpu.VMEM((NCHUNK, C), jnp.int32),
        pltpu.VMEM((C, D), jnp.float32),
        pltpu.VMEM((C, D), jnp.float32),
        pltpu.VMEM_SHARED((V, D), jnp.float32),
        pltpu.SemaphoreType.DMA,
        pltpu.SemaphoreType.DMA,
        pltpu.SemaphoreType.DMA,
        pltpu.SemaphoreType.DMA,
    ],
)
def _sc_gather(idx_hbm, table_hbm, out_hbm, idx_v, rows0, rows1, table_sh,
               sem0, sem1, wsem0, wsem1):
    sid = lax.axis_index("s")
    wid = sid * NC + lax.axis_index("c")

    @pl.when(sid == 0)
    def _():
        pltpu.sync_copy(table_hbm, table_sh)

    pltpu.sync_copy(idx_hbm.at[wid], idx_v)
    plsc.subcore_barrier()

    base = wid * ROWS_PER_W
    bufs = (rows0, rows1)
    sems = (sem0, sem1)
    wsems = (wsem0, wsem1)

    def issue(c, buf, sem):
        for g in range(C // 16):
            vec = idx_v[c, pl.ds(g * 16, 16)]
            for j in range(16):
                r = vec[j]
                pltpu.async_copy(table_sh.at[r], buf.at[g * 16 + j], sem)

    def wait_all(buf, sem):
        pltpu.make_async_copy(table_hbm.at[pl.ds(0, C)], buf, sem).wait()

    def wait_write(buf, wsem):
        pltpu.make_async_copy(buf, out_hbm.at[pl.ds(0, C)], wsem).wait()

    issue(0, rows0, sem0)

    def outer(i2, carry):
        c0 = i2 * 2
        for b in range(2):
            c = c0 + b

            @pl.when((c + 1 < NCHUNK) & (c >= 1))
            def _():
                wait_write(bufs[1 - b], wsems[1 - b])

            @pl.when(c + 1 < NCHUNK)
            def _():
                issue(c + 1, bufs[1 - b], sems[1 - b])

            wait_all(bufs[b], sems[b])
            pltpu.async_copy(bufs[b], out_hbm.at[pl.ds(base + c * C, C)],
                             wsems[b])
        return carry

    lax.fori_loop(0, NCHUNK // 2, outer, 0)
    wait_write(rows0, wsem0)
    wait_write(rows1, wsem1)


def kernel(input_ids, attention_mask, embed):
    idx = input_ids.reshape(NW, NCHUNK, C).astype(jnp.int32)
    out = _sc_gather(idx, embed)
    return out.reshape(B, S, D)
